# baseline (speedup 1.0000x reference)
import functools

import jax
import jax.numpy as jnp
from jax import lax
from jax.experimental import pallas as pl
from jax.experimental.pallas import tpu as pltpu
from jax.experimental.pallas import tpu_sc as plsc

N = 10000
E = 320000
D = 128
H = 128

NC = 2
NS = 16
LANES = 16
CHUNK = 128

N_PAD = 10240
ROWS_PER_TILE = N_PAD // NS
HALF = H // NC

EB = 2560
E_PAD = EB * CHUNK
CH_TILE = EB // NS
CH_WORKER = EB // (NC * NS)
NBUF = 4

_MESH = plsc.VectorSubcoreMesh(
    core_axis_name="c", subcore_axis_name="s", num_cores=NC, num_subcores=NS)

_SPLAT_DNUMS = lax.GatherDimensionNumbers(
    offset_dims=(), collapsed_slice_dims=(0,), start_index_map=(0,))


def _splat(vec, e):
  idx = jnp.full((LANES, 1), e, jnp.int32)
  return lax.gather(vec, idx, _SPLAT_DNUMS, (1,),
                    mode=lax.GatherScatterMode.PROMISE_IN_BOUNDS)


def _rsqrt16(x):
  xi = lax.bitcast_convert_type(x, jnp.int32)
  yi = jnp.int32(0x5F3759DF) - (xi >> 1)
  y = lax.bitcast_convert_type(yi, jnp.float32)
  for _ in range(3):
    y = y * (1.5 - 0.5 * x * y * y)
  return jnp.where(x > 0.0, y, 0.0)


@functools.partial(
    pl.kernel,
    out_type=jax.ShapeDtypeStruct((E_PAD,), jnp.float32),
    mesh=_MESH,
    compiler_params=pltpu.CompilerParams(use_tc_tiling_on_sc=False),
    scratch_types=dict(
        deg_sh=pltpu.VMEM_SHARED((N_PAD,), jnp.float32),
        dis_sh=pltpu.VMEM_SHARED((N_PAD,), jnp.float32),
        colb=pltpu.VMEM((CH_TILE, CHUNK), jnp.int32),
        ewb=pltpu.VMEM((CH_TILE * CHUNK,), jnp.float32),
        valb=pltpu.VMEM((CH_WORKER * CHUNK,), jnp.float32),
        valb2=pltpu.VMEM((CH_WORKER * CHUNK,), jnp.float32),
        nodebuf=pltpu.VMEM((ROWS_PER_TILE,), jnp.float32),
        disbuf=pltpu.VMEM((ROWS_PER_TILE,), jnp.float32),
        s_sc=pltpu.SemaphoreType.DMA,
        s_g=pltpu.SemaphoreType.DMA,
    ),
)
def _sc_prep(col2d_hbm, row2d_hbm, ew_hbm, ew2_hbm,
             deg_sh, dis_sh, colb, ewb, valb, valb2, nodebuf, disbuf,
             s_sc, s_g):
  cid = lax.axis_index("c")
  sid = lax.axis_index("s")
  node_base = sid * ROWS_PER_TILE

  def _zero(i, _):
    nodebuf[pl.ds(i * LANES, LANES)] = jnp.zeros((LANES,), jnp.float32)
    return _
  lax.fori_loop(0, ROWS_PER_TILE // LANES, _zero, None)
  pltpu.sync_copy(nodebuf, deg_sh.at[pl.ds(node_base, ROWS_PER_TILE)])

  pltpu.sync_copy(col2d_hbm.at[pl.ds(sid * CH_TILE, CH_TILE), :], colb)
  pltpu.sync_copy(ew_hbm.at[pl.ds(sid * CH_TILE * CHUNK, CH_TILE * CHUNK)],
                  ewb)
  plsc.subcore_barrier()

  def _wait_sc_one():
    pltpu.make_async_copy(ewb.at[pl.ds(0, CHUNK)], deg_sh.at[colb.at[0]],
                          s_sc).wait()

  def _deg_chunk(c, _):
    pltpu.async_copy(ewb.at[pl.ds(c * CHUNK, CHUNK)], deg_sh.at[colb.at[c]],
                     s_sc, add=True)
    @pl.when(c >= NBUF)
    def _():
      _wait_sc_one()
    return _
  lax.fori_loop(0, CH_TILE, _deg_chunk, None)
  for _ in range(NBUF):
    _wait_sc_one()
  plsc.subcore_barrier()

  pltpu.sync_copy(deg_sh.at[pl.ds(node_base, ROWS_PER_TILE)], nodebuf)
  def _dis(i, _):
    x = nodebuf[pl.ds(i * LANES, LANES)]
    disbuf[pl.ds(i * LANES, LANES)] = _rsqrt16(x)
    return _
  lax.fori_loop(0, ROWS_PER_TILE // LANES, _dis, None)
  pltpu.sync_copy(disbuf, dis_sh.at[pl.ds(node_base, ROWS_PER_TILE)])
  plsc.subcore_barrier()

  wid = cid * NS + sid
  pltpu.sync_copy(col2d_hbm.at[pl.ds(wid * CH_WORKER, CH_WORKER), :],
                  colb.at[pl.ds(0, CH_WORKER), :])
  pltpu.sync_copy(row2d_hbm.at[pl.ds(wid * CH_WORKER, CH_WORKER), :],
                  colb.at[pl.ds(CH_WORKER, CH_WORKER), :])
  pltpu.sync_copy(
      ew_hbm.at[pl.ds(wid * CH_WORKER * CHUNK, CH_WORKER * CHUNK)],
      ewb.at[pl.ds(0, CH_WORKER * CHUNK)])

  def _wait_g_one():
    pltpu.make_async_copy(dis_sh.at[colb.at[0]], valb.at[pl.ds(0, CHUNK)],
                          s_g).wait()

  def _gath_chunk(c, _):
    pltpu.async_copy(dis_sh.at[colb.at[c]], valb.at[pl.ds(c * CHUNK, CHUNK)],
                     s_g)
    pltpu.async_copy(dis_sh.at[colb.at[CH_WORKER + c]],
                     valb2.at[pl.ds(c * CHUNK, CHUNK)], s_g)
    @pl.when(c >= NBUF // 2)
    def _():
      _wait_g_one()
      _wait_g_one()
    return _
  lax.fori_loop(0, CH_WORKER, _gath_chunk, None)
  for _ in range(NBUF):
    _wait_g_one()

  def _scale(i, _):
    sl = pl.ds(i * LANES, LANES)
    valb[sl] = valb[sl] * valb2[sl] * ewb[sl]
    return _
  lax.fori_loop(0, CH_WORKER * CHUNK // LANES, _scale, None)
  pltpu.sync_copy(valb,
                  ew2_hbm.at[pl.ds(wid * CH_WORKER * CHUNK,
                                   CH_WORKER * CHUNK)])


BCH = 8
NBLK = CH_TILE // BCH

_AGG_SCRATCH = dict(
    acc_sh=pltpu.VMEM_SHARED((N_PAD, HALF), jnp.float32),
    rowA=pltpu.VMEM((BCH, CHUNK), jnp.int32),
    colA=pltpu.VMEM((BCH, CHUNK), jnp.int32),
    ewA=pltpu.VMEM((BCH * CHUNK,), jnp.float32),
    rowB=pltpu.VMEM((BCH, CHUNK), jnp.int32),
    colB=pltpu.VMEM((BCH, CHUNK), jnp.int32),
    ewB=pltpu.VMEM((BCH * CHUNK,), jnp.float32),
    msg0=pltpu.VMEM((CHUNK, HALF), jnp.float32),
    msg1=pltpu.VMEM((CHUNK, HALF), jnp.float32),
    msg2=pltpu.VMEM((CHUNK, HALF), jnp.float32),
    msg3=pltpu.VMEM((CHUNK, HALF), jnp.float32),
    out0=pltpu.VMEM((CHUNK, HALF), jnp.float32),
    out1=pltpu.VMEM((CHUNK, HALF), jnp.float32),
    out2=pltpu.VMEM((CHUNK, HALF), jnp.float32),
    out3=pltpu.VMEM((CHUNK, HALF), jnp.float32),
    sg0=pltpu.SemaphoreType.DMA,
    sg1=pltpu.SemaphoreType.DMA,
    sg2=pltpu.SemaphoreType.DMA,
    sg3=pltpu.SemaphoreType.DMA,
    ss0=pltpu.SemaphoreType.DMA,
    ss1=pltpu.SemaphoreType.DMA,
    ss2=pltpu.SemaphoreType.DMA,
    ss3=pltpu.SemaphoreType.DMA,
    s_la=pltpu.SemaphoreType.DMA,
    s_lb=pltpu.SemaphoreType.DMA,
)


@functools.partial(
    pl.kernel,
    out_type=jax.ShapeDtypeStruct((N_PAD, H), jnp.float32),
    mesh=_MESH,
    compiler_params=pltpu.CompilerParams(use_tc_tiling_on_sc=False),
    scratch_types=_AGG_SCRATCH,
)
def _sc_aggregate(t3_hbm, row2d_hbm, col2d_hbm, ew2_hbm, b_hbm, out_hbm,
                  acc_sh, rowA, colA, ewA, rowB, colB, ewB,
                  msg0, msg1, msg2, msg3, out0, out1, out2, out3,
                  sg0, sg1, sg2, sg3, ss0, ss1, ss2, ss3, s_la, s_lb):
  cid = lax.axis_index("c")
  sid = lax.axis_index("s")
  msgs = (msg0, msg1, msg2, msg3)
  outs = (out0, out1, out2, out3)
  sgs = (sg0, sg1, sg2, sg3)
  sss = (ss0, ss1, ss2, ss3)
  set_a = (rowA, colA, ewA, s_la)
  set_b = (rowB, colB, ewB, s_lb)
  rows = pl.ds(sid * ROWS_PER_TILE, ROWS_PER_TILE)
  cols = pl.ds(cid * HALF, HALF)
  tsrc = t3_hbm.at[cid]

  pltpu.sync_copy(b_hbm, ewA.at[pl.ds(0, H)])
  bvs = [ewA[pl.ds(cid * HALF + j * LANES, LANES)]
         for j in range(HALF // LANES)]
  def _bias_row(r, carry):
    for j in range(HALF // LANES):
      out0[r, pl.ds(j * LANES, LANES)] = bvs[j]
    return carry
  lax.fori_loop(0, CHUNK, _bias_row, None)
  for i in range(ROWS_PER_TILE // CHUNK):
    pltpu.sync_copy(
        out0, acc_sh.at[pl.ds(sid * ROWS_PER_TILE + i * CHUNK, CHUNK), :])
  plsc.subcore_barrier()

  def _lin_issue(st, b):
    rowX, colX, ewX, sem = st
    r0 = sid * CH_TILE + b * BCH
    pltpu.async_copy(row2d_hbm.at[pl.ds(r0, BCH), :], rowX, sem)
    pltpu.async_copy(col2d_hbm.at[pl.ds(r0, BCH), :], colX, sem)
    pltpu.async_copy(ew2_hbm.at[pl.ds(r0 * CHUNK, BCH * CHUNK)], ewX, sem)

  def _lin_wait(st):
    rowX, colX, ewX, sem = st
    pltpu.make_async_copy(row2d_hbm.at[pl.ds(0, BCH), :], rowX, sem).wait()
    pltpu.make_async_copy(col2d_hbm.at[pl.ds(0, BCH), :], colX, sem).wait()
    pltpu.make_async_copy(ew2_hbm.at[pl.ds(0, BCH * CHUNK)], ewX, sem).wait()

  def _gather_start(st, cl, m):
    pltpu.async_copy(tsrc.at[st[0].at[cl]], msgs[m], sgs[m])

  def _gather_wait(m):
    pltpu.make_async_copy(tsrc.at[rowA.at[0]], msgs[m], sgs[m]).wait()

  def _scat_start(st, cl, o):
    pltpu.async_copy(outs[o], acc_sh.at[st[1].at[cl]], sss[o], add=True)

  def _scat_wait(o):
    pltpu.make_async_copy(outs[o], acc_sh.at[colA.at[0]], sss[o]).wait()

  def _scale(st, cl, mp, op):
    m = msgs[mp]
    o = outs[op]
    ewX = st[2]
    def _g_body(g, carry):
      ew_v = ewX[pl.ds(cl * CHUNK + g * LANES, LANES)]
      for e in range(LANES):
        w = _splat(ew_v, e)
        r = g * LANES + e
        for j in range(HALF // LANES):
          sl = pl.ds(j * LANES, LANES)
          o[r, sl] = m[r, sl] * w
      return carry
    lax.fori_loop(0, CHUNK // LANES, _g_body, None)

  GD = 3
  _lin_issue(set_a, 0)
  _lin_wait(set_a)
  for c0 in range(GD):
    _gather_start(set_a, c0, c0)

  def _pair(p, _):
    for blk, (cur, nxt_set) in enumerate(((set_a, set_b), (set_b, set_a))):
      b = 2 * p + blk

      for k in range(BCH // 4):
        for u in range(4):
          cl = k * 4 + u
          c = b * BCH + cl
          mp = u
          op = u
          mn = (u + GD) % 4
          _gather_wait(mp)
          if cl + GD < BCH:
            _gather_start(cur, cl + GD, mn)
          else:
            if blk == 0:
              if u == 1:
                _lin_wait(nxt_set)
              _gather_start(nxt_set, cl + GD - BCH, mn)
            else:
              @pl.when(p < NBLK // 2 - 1)
              def _(u=u, mn=mn, cl=cl):
                if u == 1:
                  _lin_wait(nxt_set)
                _gather_start(nxt_set, cl + GD - BCH, mn)
          @pl.when(c >= 4)
          def _(op=op):
            _scat_wait(op)
          if k == 0 and u == 3:
            if blk == 0:
              _lin_issue(nxt_set, b + 1)
            else:
              @pl.when(p < NBLK // 2 - 1)
              def _():
                _lin_issue(nxt_set, b + 1)
          _scale(cur, cl, mp, op)
          _scat_start(cur, cl, op)
    return _
  lax.fori_loop(0, NBLK // 2, _pair, None)

  for o in range(4):
    _scat_wait(o)
  plsc.subcore_barrier()

  pltpu.sync_copy(acc_sh.at[rows, :], out_hbm.at[rows, cols])


_BLK = 1024


def _mm_body(relu, x_ref, w_ref, o_ref):
  x = x_ref[...]
  if relu:
    x = jnp.maximum(x, 0.0)
  c = pl.program_id(0)
  full = jnp.dot(x, w_ref[...], preferred_element_type=jnp.float32)
  o_ref[...] = jnp.where(c == 0, full[:, :HALF], full[:, HALF:])[None]


def _tc_matmul(t, w, relu):
  return pl.pallas_call(
      functools.partial(_mm_body, relu),
      out_shape=jax.ShapeDtypeStruct((NC, N_PAD, HALF), jnp.float32),
      grid=(NC, N_PAD // _BLK),
      in_specs=[
          pl.BlockSpec((_BLK, D), lambda c, i: (i, 0)),
          pl.BlockSpec((D, H), lambda c, i: (0, 0)),
      ],
      out_specs=pl.BlockSpec((1, _BLK, HALF), lambda c, i: (c, i, 0)),
  )(t, w)


def kernel(x, edge_index, edge_weight, W1, b1, W2, b2):
  row = edge_index[0]
  col = edge_index[1]
  x_pad = jnp.pad(x, ((0, N_PAD - N), (0, 0)))

  pad_idx = (jnp.arange(E_PAD - E, dtype=jnp.int32) % N)
  row2d = jnp.concatenate([row, pad_idx]).reshape(EB, CHUNK)
  col2d = jnp.concatenate([col, pad_idx]).reshape(EB, CHUNK)
  ew_flat = jnp.concatenate(
      [edge_weight, jnp.zeros((E_PAD - E,), jnp.float32)])

  ew3 = _sc_prep(col2d, row2d, ew_flat)

  t1 = _tc_matmul(x_pad, W1, relu=False)
  agg1 = _sc_aggregate(t1, row2d, col2d, ew3, b1)
  t2 = _tc_matmul(agg1, W2, relu=True)
  agg2 = _sc_aggregate(t2, row2d, col2d, ew3, b2)
  return agg2[:N]


# speedup vs baseline: 1.0042x; 1.0042x over previous
import functools

import jax
import jax.numpy as jnp
from jax import lax
from jax.experimental import pallas as pl
from jax.experimental.pallas import tpu as pltpu
from jax.experimental.pallas import tpu_sc as plsc

N = 10000
E = 320000
D = 128
H = 128

NC = 2
NS = 16
LANES = 16
CHUNK = 128

N_PAD = 10240
ROWS_PER_TILE = N_PAD // NS
HALF = H // NC

EB = 2560
E_PAD = EB * CHUNK
CH_TILE = EB // NS
CH_WORKER = EB // (NC * NS)
NBUF = 4

_MESH = plsc.VectorSubcoreMesh(
    core_axis_name="c", subcore_axis_name="s", num_cores=NC, num_subcores=NS)

_SPLAT_DNUMS = lax.GatherDimensionNumbers(
    offset_dims=(), collapsed_slice_dims=(0,), start_index_map=(0,))


def _splat(vec, e):
  idx = jnp.full((LANES, 1), e, jnp.int32)
  return lax.gather(vec, idx, _SPLAT_DNUMS, (1,),
                    mode=lax.GatherScatterMode.PROMISE_IN_BOUNDS)


def _rsqrt16(x):
  xi = lax.bitcast_convert_type(x, jnp.int32)
  yi = jnp.int32(0x5F3759DF) - (xi >> 1)
  y = lax.bitcast_convert_type(yi, jnp.float32)
  for _ in range(3):
    y = y * (1.5 - 0.5 * x * y * y)
  return jnp.where(x > 0.0, y, 0.0)


@functools.partial(
    pl.kernel,
    out_type=jax.ShapeDtypeStruct((E_PAD,), jnp.float32),
    mesh=_MESH,
    compiler_params=pltpu.CompilerParams(use_tc_tiling_on_sc=False),
    scratch_types=dict(
        deg_sh=pltpu.VMEM_SHARED((N_PAD,), jnp.float32),
        dis_sh=pltpu.VMEM_SHARED((N_PAD,), jnp.float32),
        colb=pltpu.VMEM((CH_TILE, CHUNK), jnp.int32),
        ewb=pltpu.VMEM((CH_TILE * CHUNK,), jnp.float32),
        valb=pltpu.VMEM((CH_WORKER * CHUNK,), jnp.float32),
        valb2=pltpu.VMEM((CH_WORKER * CHUNK,), jnp.float32),
        nodebuf=pltpu.VMEM((ROWS_PER_TILE,), jnp.float32),
        disbuf=pltpu.VMEM((ROWS_PER_TILE,), jnp.float32),
        s_sc=pltpu.SemaphoreType.DMA,
        s_g=pltpu.SemaphoreType.DMA,
    ),
)
def _sc_prep(col2d_hbm, row2d_hbm, ew_hbm, ew2_hbm,
             deg_sh, dis_sh, colb, ewb, valb, valb2, nodebuf, disbuf,
             s_sc, s_g):
  cid = lax.axis_index("c")
  sid = lax.axis_index("s")
  node_base = sid * ROWS_PER_TILE

  def _zero(i, _):
    nodebuf[pl.ds(i * LANES, LANES)] = jnp.zeros((LANES,), jnp.float32)
    return _
  lax.fori_loop(0, ROWS_PER_TILE // LANES, _zero, None)
  pltpu.sync_copy(nodebuf, deg_sh.at[pl.ds(node_base, ROWS_PER_TILE)])

  pltpu.sync_copy(col2d_hbm.at[pl.ds(sid * CH_TILE, CH_TILE), :], colb)
  pltpu.sync_copy(ew_hbm.at[pl.ds(sid * CH_TILE * CHUNK, CH_TILE * CHUNK)],
                  ewb)
  plsc.subcore_barrier()

  def _wait_sc_one():
    pltpu.make_async_copy(ewb.at[pl.ds(0, CHUNK)], deg_sh.at[colb.at[0]],
                          s_sc).wait()

  def _deg_chunk(c, _):
    pltpu.async_copy(ewb.at[pl.ds(c * CHUNK, CHUNK)], deg_sh.at[colb.at[c]],
                     s_sc, add=True)
    @pl.when(c >= NBUF)
    def _():
      _wait_sc_one()
    return _
  lax.fori_loop(0, CH_TILE, _deg_chunk, None)
  for _ in range(NBUF):
    _wait_sc_one()
  plsc.subcore_barrier()

  pltpu.sync_copy(deg_sh.at[pl.ds(node_base, ROWS_PER_TILE)], nodebuf)
  def _dis(i, _):
    x = nodebuf[pl.ds(i * LANES, LANES)]
    disbuf[pl.ds(i * LANES, LANES)] = _rsqrt16(x)
    return _
  lax.fori_loop(0, ROWS_PER_TILE // LANES, _dis, None)
  pltpu.sync_copy(disbuf, dis_sh.at[pl.ds(node_base, ROWS_PER_TILE)])
  plsc.subcore_barrier()

  wid = cid * NS + sid
  pltpu.sync_copy(col2d_hbm.at[pl.ds(wid * CH_WORKER, CH_WORKER), :],
                  colb.at[pl.ds(0, CH_WORKER), :])
  pltpu.sync_copy(row2d_hbm.at[pl.ds(wid * CH_WORKER, CH_WORKER), :],
                  colb.at[pl.ds(CH_WORKER, CH_WORKER), :])
  pltpu.sync_copy(
      ew_hbm.at[pl.ds(wid * CH_WORKER * CHUNK, CH_WORKER * CHUNK)],
      ewb.at[pl.ds(0, CH_WORKER * CHUNK)])

  def _wait_g_one():
    pltpu.make_async_copy(dis_sh.at[colb.at[0]], valb.at[pl.ds(0, CHUNK)],
                          s_g).wait()

  def _gath_chunk(c, _):
    pltpu.async_copy(dis_sh.at[colb.at[c]], valb.at[pl.ds(c * CHUNK, CHUNK)],
                     s_g)
    pltpu.async_copy(dis_sh.at[colb.at[CH_WORKER + c]],
                     valb2.at[pl.ds(c * CHUNK, CHUNK)], s_g)
    @pl.when(c >= NBUF // 2)
    def _():
      _wait_g_one()
      _wait_g_one()
    return _
  lax.fori_loop(0, CH_WORKER, _gath_chunk, None)
  for _ in range(NBUF):
    _wait_g_one()

  def _scale(i, _):
    sl = pl.ds(i * LANES, LANES)
    valb[sl] = valb[sl] * valb2[sl] * ewb[sl]
    return _
  lax.fori_loop(0, CH_WORKER * CHUNK // LANES, _scale, None)
  pltpu.sync_copy(valb,
                  ew2_hbm.at[pl.ds(wid * CH_WORKER * CHUNK,
                                   CH_WORKER * CHUNK)])


BCH = 8
NBLK = CH_TILE // BCH

_AGG_SCRATCH = dict(
    acc_sh=pltpu.VMEM_SHARED((N_PAD, HALF), jnp.float32),
    rowA=pltpu.VMEM((BCH, CHUNK), jnp.int32),
    colA=pltpu.VMEM((BCH, CHUNK), jnp.int32),
    ewA=pltpu.VMEM((BCH * CHUNK,), jnp.float32),
    rowB=pltpu.VMEM((BCH, CHUNK), jnp.int32),
    colB=pltpu.VMEM((BCH, CHUNK), jnp.int32),
    ewB=pltpu.VMEM((BCH * CHUNK,), jnp.float32),
    msg0=pltpu.VMEM((CHUNK, HALF), jnp.float32),
    msg1=pltpu.VMEM((CHUNK, HALF), jnp.float32),
    msg2=pltpu.VMEM((CHUNK, HALF), jnp.float32),
    msg3=pltpu.VMEM((CHUNK, HALF), jnp.float32),
    out0=pltpu.VMEM((CHUNK, HALF), jnp.float32),
    out1=pltpu.VMEM((CHUNK, HALF), jnp.float32),
    sg0=pltpu.SemaphoreType.DMA,
    sg1=pltpu.SemaphoreType.DMA,
    sg2=pltpu.SemaphoreType.DMA,
    sg3=pltpu.SemaphoreType.DMA,
    ss0=pltpu.SemaphoreType.DMA,
    ss1=pltpu.SemaphoreType.DMA,
    s_la=pltpu.SemaphoreType.DMA,
    s_lb=pltpu.SemaphoreType.DMA,
)


@functools.partial(
    pl.kernel,
    out_type=jax.ShapeDtypeStruct((N_PAD, H), jnp.float32),
    mesh=_MESH,
    compiler_params=pltpu.CompilerParams(use_tc_tiling_on_sc=False),
    scratch_types=_AGG_SCRATCH,
)
def _sc_aggregate(t3_hbm, row2d_hbm, col2d_hbm, ew2_hbm, b_hbm, out_hbm,
                  acc_sh, rowA, colA, ewA, rowB, colB, ewB,
                  msg0, msg1, msg2, msg3, out0, out1,
                  sg0, sg1, sg2, sg3, ss0, ss1, s_la, s_lb):
  cid = lax.axis_index("c")
  sid = lax.axis_index("s")
  msgs = (msg0, msg1, msg2, msg3)
  outs = (out0, out1)
  sgs = (sg0, sg1, sg2, sg3)
  sss = (ss0, ss1)
  set_a = (rowA, colA, ewA, s_la)
  set_b = (rowB, colB, ewB, s_lb)
  rows = pl.ds(sid * ROWS_PER_TILE, ROWS_PER_TILE)
  cols = pl.ds(cid * HALF, HALF)
  tsrc = t3_hbm.at[cid]

  pltpu.sync_copy(b_hbm, ewA.at[pl.ds(0, H)])
  bvs = [ewA[pl.ds(cid * HALF + j * LANES, LANES)]
         for j in range(HALF // LANES)]
  def _bias_row(r, carry):
    for j in range(HALF // LANES):
      out0[r, pl.ds(j * LANES, LANES)] = bvs[j]
    return carry
  lax.fori_loop(0, CHUNK, _bias_row, None)
  for i in range(ROWS_PER_TILE // CHUNK):
    pltpu.sync_copy(
        out0, acc_sh.at[pl.ds(sid * ROWS_PER_TILE + i * CHUNK, CHUNK), :])
  plsc.subcore_barrier()

  def _lin_issue(st, b):
    rowX, colX, ewX, sem = st
    r0 = sid * CH_TILE + b * BCH
    pltpu.async_copy(row2d_hbm.at[pl.ds(r0, BCH), :], rowX, sem)
    pltpu.async_copy(col2d_hbm.at[pl.ds(r0, BCH), :], colX, sem)
    pltpu.async_copy(ew2_hbm.at[pl.ds(r0 * CHUNK, BCH * CHUNK)], ewX, sem)

  def _lin_wait(st):
    rowX, colX, ewX, sem = st
    pltpu.make_async_copy(row2d_hbm.at[pl.ds(0, BCH), :], rowX, sem).wait()
    pltpu.make_async_copy(col2d_hbm.at[pl.ds(0, BCH), :], colX, sem).wait()
    pltpu.make_async_copy(ew2_hbm.at[pl.ds(0, BCH * CHUNK)], ewX, sem).wait()

  def _gather_start(st, cl, m):
    pltpu.async_copy(tsrc.at[st[0].at[cl]], msgs[m], sgs[m])

  def _gather_wait(m):
    pltpu.make_async_copy(tsrc.at[rowA.at[0]], msgs[m], sgs[m]).wait()

  def _scat_start(st, cl, o):
    pltpu.async_copy(outs[o], acc_sh.at[st[1].at[cl]], sss[o], add=True)

  def _scat_wait(o):
    pltpu.make_async_copy(outs[o], acc_sh.at[colA.at[0]], sss[o]).wait()

  def _scale(st, cl, mp, op):
    m = msgs[mp]
    o = outs[op]
    ewX = st[2]
    def _g_body(g, carry):
      ew_v = ewX[pl.ds(cl * CHUNK + g * LANES, LANES)]
      for e in range(LANES):
        w = _splat(ew_v, e)
        r = g * LANES + e
        for j in range(HALF // LANES):
          sl = pl.ds(j * LANES, LANES)
          o[r, sl] = m[r, sl] * w
      return carry
    lax.fori_loop(0, CHUNK // LANES, _g_body, None)

  GD = 3
  _lin_issue(set_a, 0)
  _lin_wait(set_a)
  for c0 in range(GD):
    _gather_start(set_a, c0, c0)

  def _pair(p, _):
    for blk, (cur, nxt_set) in enumerate(((set_a, set_b), (set_b, set_a))):
      b = 2 * p + blk

      for k in range(BCH // 4):
        for u in range(4):
          cl = k * 4 + u
          c = b * BCH + cl
          mp = u
          op = u % 2
          mn = (u + GD) % 4
          _gather_wait(mp)
          if cl + GD < BCH:
            _gather_start(cur, cl + GD, mn)
          else:
            if blk == 0:
              if u == 1:
                _lin_wait(nxt_set)
              _gather_start(nxt_set, cl + GD - BCH, mn)
            else:
              @pl.when(p < NBLK // 2 - 1)
              def _(u=u, mn=mn, cl=cl):
                if u == 1:
                  _lin_wait(nxt_set)
                _gather_start(nxt_set, cl + GD - BCH, mn)
          @pl.when(c >= 2)
          def _(op=op):
            _scat_wait(op)
          if k == 0 and u == 1:
            if blk == 0:
              _lin_issue(nxt_set, b + 1)
            else:
              @pl.when(p < NBLK // 2 - 1)
              def _():
                _lin_issue(nxt_set, b + 1)
          _scale(cur, cl, mp, op)
          _scat_start(cur, cl, op)
    return _
  lax.fori_loop(0, NBLK // 2, _pair, None)

  _scat_wait(0)
  _scat_wait(1)
  plsc.subcore_barrier()

  pltpu.sync_copy(acc_sh.at[rows, :], out_hbm.at[rows, cols])


_BLK = 1024


def _mm_body(relu, x_ref, w_ref, o_ref):
  x = x_ref[...]
  if relu:
    x = jnp.maximum(x, 0.0)
  c = pl.program_id(0)
  full = jnp.dot(x, w_ref[...], preferred_element_type=jnp.float32)
  o_ref[...] = jnp.where(c == 0, full[:, :HALF], full[:, HALF:])[None]


def _tc_matmul(t, w, relu):
  return pl.pallas_call(
      functools.partial(_mm_body, relu),
      out_shape=jax.ShapeDtypeStruct((NC, N_PAD, HALF), jnp.float32),
      grid=(NC, N_PAD // _BLK),
      in_specs=[
          pl.BlockSpec((_BLK, D), lambda c, i: (i, 0)),
          pl.BlockSpec((D, H), lambda c, i: (0, 0)),
      ],
      out_specs=pl.BlockSpec((1, _BLK, HALF), lambda c, i: (c, i, 0)),
  )(t, w)


def kernel(x, edge_index, edge_weight, W1, b1, W2, b2):
  row = edge_index[0]
  col = edge_index[1]
  x_pad = jnp.pad(x, ((0, N_PAD - N), (0, 0)))

  pad_idx = (jnp.arange(E_PAD - E, dtype=jnp.int32) % N)
  row2d = jnp.concatenate([row, pad_idx]).reshape(EB, CHUNK)
  col2d = jnp.concatenate([col, pad_idx]).reshape(EB, CHUNK)
  ew_flat = jnp.concatenate(
      [edge_weight, jnp.zeros((E_PAD - E,), jnp.float32)])

  ew3 = _sc_prep(col2d, row2d, ew_flat)

  t1 = _tc_matmul(x_pad, W1, relu=False)
  agg1 = _sc_aggregate(t1, row2d, col2d, ew3, b1)
  t2 = _tc_matmul(agg1, W2, relu=True)
  agg2 = _sc_aggregate(t2, row2d, col2d, ew3, b2)
  return agg2[:N]


# speedup vs baseline: 1.0121x; 1.0079x over previous
import functools

import jax
import jax.numpy as jnp
from jax import lax
from jax.experimental import pallas as pl
from jax.experimental.pallas import tpu as pltpu
from jax.experimental.pallas import tpu_sc as plsc

N = 10000
E = 320000
D = 128
H = 128

NC = 2
NS = 16
LANES = 16
CHUNK = 128

N_PAD = 10240
ROWS_PER_TILE = N_PAD // NS
HALF = H // NC

EB = 2560
E_PAD = EB * CHUNK
CH_TILE = EB // NS
CH_WORKER = EB // (NC * NS)
NBUF = 4

_MESH = plsc.VectorSubcoreMesh(
    core_axis_name="c", subcore_axis_name="s", num_cores=NC, num_subcores=NS)

_SPLAT_DNUMS = lax.GatherDimensionNumbers(
    offset_dims=(), collapsed_slice_dims=(0,), start_index_map=(0,))


def _splat(vec, e):
  idx = jnp.full((LANES, 1), e, jnp.int32)
  return lax.gather(vec, idx, _SPLAT_DNUMS, (1,),
                    mode=lax.GatherScatterMode.PROMISE_IN_BOUNDS)


def _rsqrt16(x):
  xi = lax.bitcast_convert_type(x, jnp.int32)
  yi = jnp.int32(0x5F3759DF) - (xi >> 1)
  y = lax.bitcast_convert_type(yi, jnp.float32)
  for _ in range(3):
    y = y * (1.5 - 0.5 * x * y * y)
  return jnp.where(x > 0.0, y, 0.0)


@functools.partial(
    pl.kernel,
    out_type=jax.ShapeDtypeStruct((E_PAD,), jnp.float32),
    mesh=_MESH,
    compiler_params=pltpu.CompilerParams(use_tc_tiling_on_sc=False),
    scratch_types=dict(
        deg_sh=pltpu.VMEM_SHARED((N_PAD,), jnp.float32),
        dis_sh=pltpu.VMEM_SHARED((N_PAD,), jnp.float32),
        colb=pltpu.VMEM((CH_TILE, CHUNK), jnp.int32),
        ewb=pltpu.VMEM((CH_TILE * CHUNK,), jnp.float32),
        valb=pltpu.VMEM((CH_WORKER * CHUNK,), jnp.float32),
        valb2=pltpu.VMEM((CH_WORKER * CHUNK,), jnp.float32),
        nodebuf=pltpu.VMEM((ROWS_PER_TILE,), jnp.float32),
        disbuf=pltpu.VMEM((ROWS_PER_TILE,), jnp.float32),
        s_sc=pltpu.SemaphoreType.DMA,
        s_g=pltpu.SemaphoreType.DMA,
    ),
)
def _sc_prep(col2d_hbm, row2d_hbm, ew_hbm, ew2_hbm,
             deg_sh, dis_sh, colb, ewb, valb, valb2, nodebuf, disbuf,
             s_sc, s_g):
  cid = lax.axis_index("c")
  sid = lax.axis_index("s")
  node_base = sid * ROWS_PER_TILE

  def _zero(i, _):
    nodebuf[pl.ds(i * LANES, LANES)] = jnp.zeros((LANES,), jnp.float32)
    return _
  lax.fori_loop(0, ROWS_PER_TILE // LANES, _zero, None)
  pltpu.sync_copy(nodebuf, deg_sh.at[pl.ds(node_base, ROWS_PER_TILE)])

  pltpu.sync_copy(col2d_hbm.at[pl.ds(sid * CH_TILE, CH_TILE), :], colb)
  pltpu.sync_copy(ew_hbm.at[pl.ds(sid * CH_TILE * CHUNK, CH_TILE * CHUNK)],
                  ewb)
  plsc.subcore_barrier()

  def _wait_sc_one():
    pltpu.make_async_copy(ewb.at[pl.ds(0, CHUNK)], deg_sh.at[colb.at[0]],
                          s_sc).wait()

  DEGQ = 12
  def _deg_chunk(c, _):
    pltpu.async_copy(ewb.at[pl.ds(c * CHUNK, CHUNK)], deg_sh.at[colb.at[c]],
                     s_sc, add=True)
    @pl.when(c >= DEGQ)
    def _():
      _wait_sc_one()
    return _
  lax.fori_loop(0, CH_TILE, _deg_chunk, None)
  for _ in range(DEGQ):
    _wait_sc_one()
  plsc.subcore_barrier()

  pltpu.sync_copy(deg_sh.at[pl.ds(node_base, ROWS_PER_TILE)], nodebuf)
  def _dis(i, _):
    x = nodebuf[pl.ds(i * LANES, LANES)]
    disbuf[pl.ds(i * LANES, LANES)] = _rsqrt16(x)
    return _
  lax.fori_loop(0, ROWS_PER_TILE // LANES, _dis, None)
  pltpu.sync_copy(disbuf, dis_sh.at[pl.ds(node_base, ROWS_PER_TILE)])
  plsc.subcore_barrier()

  wid = cid * NS + sid
  pltpu.sync_copy(col2d_hbm.at[pl.ds(wid * CH_WORKER, CH_WORKER), :],
                  colb.at[pl.ds(0, CH_WORKER), :])
  pltpu.sync_copy(row2d_hbm.at[pl.ds(wid * CH_WORKER, CH_WORKER), :],
                  colb.at[pl.ds(CH_WORKER, CH_WORKER), :])
  pltpu.sync_copy(
      ew_hbm.at[pl.ds(wid * CH_WORKER * CHUNK, CH_WORKER * CHUNK)],
      ewb.at[pl.ds(0, CH_WORKER * CHUNK)])

  def _wait_g_one():
    pltpu.make_async_copy(dis_sh.at[colb.at[0]], valb.at[pl.ds(0, CHUNK)],
                          s_g).wait()

  def _gath_chunk(c, _):
    pltpu.async_copy(dis_sh.at[colb.at[c]], valb.at[pl.ds(c * CHUNK, CHUNK)],
                     s_g)
    pltpu.async_copy(dis_sh.at[colb.at[CH_WORKER + c]],
                     valb2.at[pl.ds(c * CHUNK, CHUNK)], s_g)
    @pl.when(c >= 6)
    def _():
      _wait_g_one()
      _wait_g_one()
    return _
  lax.fori_loop(0, CH_WORKER, _gath_chunk, None)
  for _ in range(12):
    _wait_g_one()

  def _scale(i, _):
    sl = pl.ds(i * LANES, LANES)
    valb[sl] = valb[sl] * valb2[sl] * ewb[sl]
    return _
  lax.fori_loop(0, CH_WORKER * CHUNK // LANES, _scale, None)
  pltpu.sync_copy(valb,
                  ew2_hbm.at[pl.ds(wid * CH_WORKER * CHUNK,
                                   CH_WORKER * CHUNK)])


BCH = 8
NBLK = CH_TILE // BCH

_AGG_SCRATCH = dict(
    acc_sh=pltpu.VMEM_SHARED((N_PAD, HALF), jnp.float32),
    rowA=pltpu.VMEM((BCH, CHUNK), jnp.int32),
    colA=pltpu.VMEM((BCH, CHUNK), jnp.int32),
    ewA=pltpu.VMEM((BCH * CHUNK,), jnp.float32),
    rowB=pltpu.VMEM((BCH, CHUNK), jnp.int32),
    colB=pltpu.VMEM((BCH, CHUNK), jnp.int32),
    ewB=pltpu.VMEM((BCH * CHUNK,), jnp.float32),
    msg0=pltpu.VMEM((CHUNK, HALF), jnp.float32),
    msg1=pltpu.VMEM((CHUNK, HALF), jnp.float32),
    msg2=pltpu.VMEM((CHUNK, HALF), jnp.float32),
    msg3=pltpu.VMEM((CHUNK, HALF), jnp.float32),
    out0=pltpu.VMEM((CHUNK, HALF), jnp.float32),
    out1=pltpu.VMEM((CHUNK, HALF), jnp.float32),
    sg0=pltpu.SemaphoreType.DMA,
    sg1=pltpu.SemaphoreType.DMA,
    sg2=pltpu.SemaphoreType.DMA,
    sg3=pltpu.SemaphoreType.DMA,
    ss0=pltpu.SemaphoreType.DMA,
    ss1=pltpu.SemaphoreType.DMA,
    s_la=pltpu.SemaphoreType.DMA,
    s_lb=pltpu.SemaphoreType.DMA,
)


@functools.partial(
    pl.kernel,
    out_type=jax.ShapeDtypeStruct((N_PAD, H), jnp.float32),
    mesh=_MESH,
    compiler_params=pltpu.CompilerParams(use_tc_tiling_on_sc=False),
    scratch_types=_AGG_SCRATCH,
)
def _sc_aggregate(t3_hbm, row2d_hbm, col2d_hbm, ew2_hbm, b_hbm, out_hbm,
                  acc_sh, rowA, colA, ewA, rowB, colB, ewB,
                  msg0, msg1, msg2, msg3, out0, out1,
                  sg0, sg1, sg2, sg3, ss0, ss1, s_la, s_lb):
  cid = lax.axis_index("c")
  sid = lax.axis_index("s")
  msgs = (msg0, msg1, msg2, msg3)
  outs = (out0, out1)
  sgs = (sg0, sg1, sg2, sg3)
  sss = (ss0, ss1)
  set_a = (rowA, colA, ewA, s_la)
  set_b = (rowB, colB, ewB, s_lb)
  rows = pl.ds(sid * ROWS_PER_TILE, ROWS_PER_TILE)
  cols = pl.ds(cid * HALF, HALF)
  tsrc = t3_hbm.at[cid]

  pltpu.sync_copy(b_hbm, ewA.at[pl.ds(0, H)])
  bvs = [ewA[pl.ds(cid * HALF + j * LANES, LANES)]
         for j in range(HALF // LANES)]
  def _bias_row(r, carry):
    for j in range(HALF // LANES):
      out0[r, pl.ds(j * LANES, LANES)] = bvs[j]
    return carry
  lax.fori_loop(0, CHUNK, _bias_row, None)
  for i in range(ROWS_PER_TILE // CHUNK):
    pltpu.sync_copy(
        out0, acc_sh.at[pl.ds(sid * ROWS_PER_TILE + i * CHUNK, CHUNK), :])
  plsc.subcore_barrier()

  def _lin_issue(st, b):
    rowX, colX, ewX, sem = st
    r0 = sid * CH_TILE + b * BCH
    pltpu.async_copy(row2d_hbm.at[pl.ds(r0, BCH), :], rowX, sem)
    pltpu.async_copy(col2d_hbm.at[pl.ds(r0, BCH), :], colX, sem)
    pltpu.async_copy(ew2_hbm.at[pl.ds(r0 * CHUNK, BCH * CHUNK)], ewX, sem)

  def _lin_wait(st):
    rowX, colX, ewX, sem = st
    pltpu.make_async_copy(row2d_hbm.at[pl.ds(0, BCH), :], rowX, sem).wait()
    pltpu.make_async_copy(col2d_hbm.at[pl.ds(0, BCH), :], colX, sem).wait()
    pltpu.make_async_copy(ew2_hbm.at[pl.ds(0, BCH * CHUNK)], ewX, sem).wait()

  def _gather_start(st, cl, m):
    pltpu.async_copy(tsrc.at[st[0].at[cl]], msgs[m], sgs[m])

  def _gather_wait(m):
    pltpu.make_async_copy(tsrc.at[rowA.at[0]], msgs[m], sgs[m]).wait()

  def _scat_start(st, cl, o):
    pltpu.async_copy(outs[o], acc_sh.at[st[1].at[cl]], sss[o], add=True)

  def _scat_wait(o):
    pltpu.make_async_copy(outs[o], acc_sh.at[colA.at[0]], sss[o]).wait()

  def _scale(st, cl, mp, op):
    m = msgs[mp]
    o = outs[op]
    ewX = st[2]
    def _g_body(g, carry):
      ew_v = ewX[pl.ds(cl * CHUNK + g * LANES, LANES)]
      for e in range(LANES):
        w = _splat(ew_v, e)
        r = g * LANES + e
        for j in range(HALF // LANES):
          sl = pl.ds(j * LANES, LANES)
          o[r, sl] = m[r, sl] * w
      return carry
    lax.fori_loop(0, CHUNK // LANES, _g_body, None)

  GD = 3
  _lin_issue(set_a, 0)
  _lin_wait(set_a)
  for c0 in range(GD):
    _gather_start(set_a, c0, c0)

  def _pair(p, _):
    for blk, (cur, nxt_set) in enumerate(((set_a, set_b), (set_b, set_a))):
      b = 2 * p + blk

      for k in range(BCH // 4):
        for u in range(4):
          cl = k * 4 + u
          c = b * BCH + cl
          mp = u
          op = u % 2
          mn = (u + GD) % 4
          _gather_wait(mp)
          if cl + GD < BCH:
            _gather_start(cur, cl + GD, mn)
          else:
            if blk == 0:
              if u == 1:
                _lin_wait(nxt_set)
              _gather_start(nxt_set, cl + GD - BCH, mn)
            else:
              @pl.when(p < NBLK // 2 - 1)
              def _(u=u, mn=mn, cl=cl):
                if u == 1:
                  _lin_wait(nxt_set)
                _gather_start(nxt_set, cl + GD - BCH, mn)
          @pl.when(c >= 2)
          def _(op=op):
            _scat_wait(op)
          if k == 0 and u == 1:
            if blk == 0:
              _lin_issue(nxt_set, b + 1)
            else:
              @pl.when(p < NBLK // 2 - 1)
              def _():
                _lin_issue(nxt_set, b + 1)
          _scale(cur, cl, mp, op)
          _scat_start(cur, cl, op)
    return _
  lax.fori_loop(0, NBLK // 2, _pair, None)

  _scat_wait(0)
  _scat_wait(1)
  plsc.subcore_barrier()

  pltpu.sync_copy(acc_sh.at[rows, :], out_hbm.at[rows, cols])


_BLK = 1024


def _mm_body(relu, x_ref, w_ref, o_ref):
  x = x_ref[...]
  if relu:
    x = jnp.maximum(x, 0.0)
  c = pl.program_id(0)
  full = jnp.dot(x, w_ref[...], preferred_element_type=jnp.float32)
  o_ref[...] = jnp.where(c == 0, full[:, :HALF], full[:, HALF:])[None]


def _tc_matmul(t, w, relu):
  return pl.pallas_call(
      functools.partial(_mm_body, relu),
      out_shape=jax.ShapeDtypeStruct((NC, N_PAD, HALF), jnp.float32),
      grid=(NC, N_PAD // _BLK),
      in_specs=[
          pl.BlockSpec((_BLK, D), lambda c, i: (i, 0)),
          pl.BlockSpec((D, H), lambda c, i: (0, 0)),
      ],
      out_specs=pl.BlockSpec((1, _BLK, HALF), lambda c, i: (c, i, 0)),
  )(t, w)


def kernel(x, edge_index, edge_weight, W1, b1, W2, b2):
  row = edge_index[0]
  col = edge_index[1]
  x_pad = jnp.pad(x, ((0, N_PAD - N), (0, 0)))

  pad_idx = (jnp.arange(E_PAD - E, dtype=jnp.int32) % N)
  row2d = jnp.concatenate([row, pad_idx]).reshape(EB, CHUNK)
  col2d = jnp.concatenate([col, pad_idx]).reshape(EB, CHUNK)
  ew_flat = jnp.concatenate(
      [edge_weight, jnp.zeros((E_PAD - E,), jnp.float32)])

  ew3 = _sc_prep(col2d, row2d, ew_flat)

  t1 = _tc_matmul(x_pad, W1, relu=False)
  agg1 = _sc_aggregate(t1, row2d, col2d, ew3, b1)
  t2 = _tc_matmul(agg1, W2, relu=True)
  agg2 = _sc_aggregate(t2, row2d, col2d, ew3, b2)
  return agg2[:N]
